# baseline (speedup 1.0000x reference)
import functools
import jax
import jax.numpy as jnp
from jax import lax
from jax.experimental import pallas as pl
from jax.experimental.pallas import tpu as pltpu
from jax.experimental.pallas import tpu_sc as plsc


def kernel(inputs, pembs_weight):
    batch_size, seqs_len = inputs.shape[:2]
    num_units = pembs_weight.shape[1]
    table = pembs_weight[:seqs_len]

    NC, NS = 2, 16
    NW = NC * NS
    b_per_w = batch_size // NW
    REP = 4
    n_bursts = b_per_w // REP

    mesh = plsc.VectorSubcoreMesh(core_axis_name="c", subcore_axis_name="s")

    @functools.partial(
        pl.kernel,
        mesh=mesh,
        out_type=jax.ShapeDtypeStruct((batch_size, seqs_len, num_units), jnp.float32),
        scratch_types=[
            pltpu.VMEM((REP, seqs_len, num_units), jnp.float32),
        ],
    )
    def k(table_hbm, out_hbm, tab_v):
        wid = lax.axis_index("s") * NC + lax.axis_index("c")
        base = wid * b_per_w
        for j in range(REP):
            pltpu.sync_copy(table_hbm, tab_v.at[j])

        def body(i, carry):
            pltpu.sync_copy(tab_v, out_hbm.at[pl.ds(base + i * REP, REP)])
            return carry

        lax.fori_loop(0, n_bursts, body, 0)

    return k(table)


# speedup vs baseline: 1.0968x; 1.0968x over previous
import functools
import jax
import jax.numpy as jnp
from jax import lax
from jax.experimental import pallas as pl
from jax.experimental.pallas import tpu as pltpu
from jax.experimental.pallas import tpu_sc as plsc


def kernel(inputs, pembs_weight):
    batch_size, seqs_len = inputs.shape[:2]
    num_units = pembs_weight.shape[1]
    table = pembs_weight[:seqs_len]

    NC, NS = 2, 16
    NW = NC * NS
    b_per_w = batch_size // NW

    mesh = plsc.VectorSubcoreMesh(core_axis_name="c", subcore_axis_name="s")

    @functools.partial(
        pl.kernel,
        mesh=mesh,
        out_type=jax.ShapeDtypeStruct((batch_size, seqs_len, num_units), jnp.float32),
        scratch_types=[pltpu.VMEM((seqs_len, num_units), jnp.float32)],
    )
    def k(table_hbm, out_hbm, tab_v):
        wid = lax.axis_index("s") * NC + lax.axis_index("c")
        base = wid * b_per_w
        pltpu.sync_copy(table_hbm, tab_v)

        def body(i, carry):
            pltpu.sync_copy(tab_v, out_hbm.at[base + i])
            return carry

        lax.fori_loop(0, b_per_w, body, 0)

    return k(table)
